# baseline (speedup 1.0000x reference)
import jax
import jax.numpy as jnp
from jax import lax
from jax.experimental import pallas as pl
from jax.experimental.pallas import tpu as pltpu
from jax.experimental.pallas import tpu_sc as plsc

BATCH = 16
NINST = 128
TIME = 8192
LST = 64
LANES = 16

NCORES = 2
HALF = TIME // 2

W = 256
NCHUNK = HALF // W
NFR = W // LST
NRG = NINST // LANES

NBIS = 4
NMIC = 2
UNR = 32


def _relu(v):
    return jnp.maximum(v, 0.0)


def _zeros():
    return jnp.zeros((LANES,), jnp.float32)


def _sc_body(x_hbm, out_hbm, buf0, buf1, tau_i,
             sin0, sin1, sout0, sout1):
    wid = lax.axis_index("s") * NCORES + lax.axis_index("c")
    b = wid // 2
    h = wid % 2
    iota = lax.iota(jnp.int32, LANES)
    coff = [iota + u for u in range(UNR)]

    def hbm_at(ci):
        return x_hbm.at[b, :, pl.ds(h * HALF + ci * W, W)]

    def out_at(ci):
        return out_hbm.at[b, :, pl.ds(h * HALF + ci * W, W)]

    def compute(buf):
        def inst_cg(cg, carry2):
            c0 = cg * LANES

            def col(r16, u):
                return buf[r16 * UNR + u, pl.ds(c0, LANES)]

            def maxu(r16, m):
                for u in range(UNR):
                    m = jnp.maximum(m, col(r16, u))
                return m

            m = lax.fori_loop(1, NINST // UNR, maxu, maxu(0, col(0, 0)))

            def bis(_, lh):
                lo, hi = lh
                mid = lo + 0.5 * (hi - lo)

                def gsum(r16, g):
                    for u in range(UNR):
                        g = g + _relu(col(r16, u) - mid)
                    return g

                g = lax.fori_loop(0, NINST // UNR, gsum, _zeros())
                ge = g >= 1.0
                return jnp.where(ge, mid, lo), jnp.where(ge, hi, mid)

            lo, _ = lax.fori_loop(0, NBIS, bis, (m - 1.0, m))

            def mic(_, tau):
                def acc(r16, gk):
                    g, k = gk
                    for u in range(UNR):
                        d = col(r16, u) - tau
                        g = g + _relu(d)
                        k = k + jnp.where(d > 0.0, 1.0, 0.0)
                    return g, k

                g, k = lax.fori_loop(0, NINST // UNR, acc,
                                     (_zeros(), _zeros()))
                return tau + (g - 1.0) / k

            tau_i[pl.ds(c0, LANES)] = lax.fori_loop(0, NMIC, mic, lo)
            return carry2

        lax.fori_loop(0, W // LANES, inst_cg, 0)

        def time_frg(frg, carry2):
            f = frg // NRG
            rg = frg % NRG
            rows = rg * LANES + iota
            fcol = jnp.full((LANES,), f * LST, jnp.int32)

            def phases(j16):
                jb = jnp.full((LANES,), j16 * UNR, jnp.int32)
                return [(jb + coff[u]) & (LST - 1) for u in range(UNR)]

            def gat(t):
                return plsc.load_gather(buf, [rows, fcol + t])

            def maxu(j16, m):
                for t in phases(j16):
                    m = jnp.maximum(m, gat(t))
                return m

            m = lax.fori_loop(1, LST // UNR, maxu, maxu(0, gat(coff[0])))

            def bis(_, lh):
                lo, hi = lh
                mid = lo + 0.5 * (hi - lo)

                def gsum(j16, g):
                    for t in phases(j16):
                        g = g + _relu(gat(t) - mid)
                    return g

                g = lax.fori_loop(0, LST // UNR, gsum, _zeros())
                ge = g >= 1.0
                return jnp.where(ge, mid, lo), jnp.where(ge, hi, mid)

            lo, _ = lax.fori_loop(0, NBIS, bis, (m - 1.0, m))

            def mic(_, tau):
                def acc(j16, gk):
                    g, k = gk
                    for t in phases(j16):
                        d = gat(t) - tau
                        g = g + _relu(d)
                        k = k + jnp.where(d > 0.0, 1.0, 0.0)
                    return g, k

                g, k = lax.fori_loop(0, LST // UNR, acc,
                                     (_zeros(), _zeros()))
                return tau + (g - 1.0) / k

            tau_t = lax.fori_loop(0, NMIC, mic, lo)

            def outj(j16, carry3):
                for t in phases(j16):
                    tc = fcol + t
                    z = plsc.load_gather(buf, [rows, tc])
                    ti = plsc.load_gather(tau_i, [tc])
                    plsc.store_scatter(buf, [rows, tc],
                                       _relu(z - ti) * _relu(z - tau_t))
                return carry3

            lax.fori_loop(0, LST // UNR, outj, 0)
            return carry2

        lax.fori_loop(0, NFR * NRG, time_frg, 0)

    pltpu.async_copy(hbm_at(0), buf0, sin0)

    def pair(j, carry):
        ci0 = 2 * j
        ci1 = ci0 + 1

        @pl.when(j > 0)
        def _():
            pltpu.make_async_copy(buf1, out_at(ci1 - 2), sout1).wait()

        pltpu.async_copy(hbm_at(ci1), buf1, sin1)
        pltpu.make_async_copy(hbm_at(ci0), buf0, sin0).wait()
        compute(buf0)
        pltpu.async_copy(buf0, out_at(ci0), sout0)

        pltpu.make_async_copy(hbm_at(ci1), buf1, sin1).wait()
        compute(buf1)
        pltpu.async_copy(buf1, out_at(ci1), sout1)

        @pl.when(ci0 + 2 < NCHUNK)
        def _():
            pltpu.make_async_copy(buf0, out_at(ci0), sout0).wait()
            pltpu.async_copy(hbm_at(ci0 + 2), buf0, sin0)

        return carry

    lax.fori_loop(0, NCHUNK // 2, pair, 0)
    pltpu.make_async_copy(buf0, out_at(NCHUNK - 2), sout0).wait()
    pltpu.make_async_copy(buf1, out_at(NCHUNK - 1), sout1).wait()


@jax.jit
def kernel(midis_out):
    mesh = plsc.VectorSubcoreMesh(core_axis_name="c", subcore_axis_name="s")
    fn = pl.kernel(
        _sc_body,
        out_type=jax.ShapeDtypeStruct((BATCH, NINST, TIME), jnp.float32),
        mesh=mesh,
        compiler_params=pltpu.CompilerParams(needs_layout_passes=False),
        scratch_types=[
            pltpu.VMEM((NINST, W), jnp.float32),
            pltpu.VMEM((NINST, W), jnp.float32),
            pltpu.VMEM((W,), jnp.float32),
            pltpu.SemaphoreType.DMA,
            pltpu.SemaphoreType.DMA,
            pltpu.SemaphoreType.DMA,
            pltpu.SemaphoreType.DMA,
        ],
    )
    return fn(midis_out)


# speedup vs baseline: 1.0590x; 1.0590x over previous
import jax
import jax.numpy as jnp
from jax import lax
from jax.experimental import pallas as pl
from jax.experimental.pallas import tpu as pltpu
from jax.experimental.pallas import tpu_sc as plsc

BATCH = 16
NINST = 128
TIME = 8192
LST = 64
LANES = 16

NCORES = 2
HALF = TIME // 2

W = 256
NCHUNK = HALF // W
NFR = W // LST
NRG = NINST // LANES

NBIS = 4
NMIC = 2
UNR = 16


def _relu(v):
    return jnp.maximum(v, 0.0)


def _zeros():
    return jnp.zeros((LANES,), jnp.float32)


def _sc_body(x_hbm, out_hbm, buf0, buf1, tau_i, tau_s,
             sin0, sin1, sout0, sout1):
    wid = lax.axis_index("s") * NCORES + lax.axis_index("c")
    b = wid // 2
    h = wid % 2
    iota = lax.iota(jnp.int32, LANES)
    coff = [iota + u for u in range(UNR)]

    def hbm_at(ci):
        return x_hbm.at[b, :, pl.ds(h * HALF + ci * W, W)]

    def out_at(ci):
        return out_hbm.at[b, :, pl.ds(h * HALF + ci * W, W)]

    def compute(buf):
        def inst_cg(cg, carry2):
            c0 = cg * LANES

            def col(r16, u):
                return buf[r16 * UNR + u, pl.ds(c0, LANES)]

            def maxu(r16, m):
                for u in range(UNR):
                    m = jnp.maximum(m, col(r16, u))
                return m

            m = lax.fori_loop(1, NINST // UNR, maxu, maxu(0, col(0, 0)))

            def bis(_, lh):
                lo, hi = lh
                mid = lo + 0.5 * (hi - lo)

                def gsum(r16, g):
                    for u in range(UNR):
                        g = g + _relu(col(r16, u) - mid)
                    return g

                g = lax.fori_loop(0, NINST // UNR, gsum, _zeros())
                ge = g >= 1.0
                return jnp.where(ge, mid, lo), jnp.where(ge, hi, mid)

            lo, _ = lax.fori_loop(0, NBIS, bis, (m - 1.0, m))

            def mic(_, tau):
                def acc(r16, gk):
                    g, k = gk
                    for u in range(UNR):
                        d = col(r16, u) - tau
                        g = g + _relu(d)
                        k = k + jnp.where(d > 0.0, 1.0, 0.0)
                    return g, k

                g, k = lax.fori_loop(0, NINST // UNR, acc,
                                     (_zeros(), _zeros()))
                return tau + (g - 1.0) / k

            tau_i[pl.ds(c0, LANES)] = lax.fori_loop(0, NMIC, mic, lo)
            return carry2

        lax.fori_loop(0, W // LANES, inst_cg, 0)

        def time_frg(frg, carry2):
            f = frg // NRG
            rg = frg % NRG
            rows = rg * LANES + iota
            fcol = jnp.full((LANES,), f * LST, jnp.int32)

            def phases(j16):
                jb = jnp.full((LANES,), j16 * UNR, jnp.int32)
                return [(jb + coff[u]) & (LST - 1) for u in range(UNR)]

            def gat(t):
                return plsc.load_gather(buf, [rows, fcol + t])

            def maxu(j16, m):
                for t in phases(j16):
                    m = jnp.maximum(m, gat(t))
                return m

            m = lax.fori_loop(1, LST // UNR, maxu, maxu(0, gat(coff[0])))

            def bis(_, lh):
                lo, hi = lh
                mid = lo + 0.5 * (hi - lo)

                def gsum(j16, g):
                    for t in phases(j16):
                        g = g + _relu(gat(t) - mid)
                    return g

                g = lax.fori_loop(0, LST // UNR, gsum, _zeros())
                ge = g >= 1.0
                return jnp.where(ge, mid, lo), jnp.where(ge, hi, mid)

            lo, _ = lax.fori_loop(0, NBIS, bis, (m - 1.0, m))

            def mic(_, tau):
                def acc(j16, gk):
                    g, k = gk
                    for t in phases(j16):
                        d = gat(t) - tau
                        g = g + _relu(d)
                        k = k + jnp.where(d > 0.0, 1.0, 0.0)
                    return g, k

                g, k = lax.fori_loop(0, LST // UNR, acc,
                                     (_zeros(), _zeros()))
                return tau + (g - 1.0) / k

            tau_t = lax.fori_loop(0, NMIC, mic, lo)
            tau_s[pl.ds(f * NINST + rg * LANES, LANES)] = tau_t
            return carry2

        lax.fori_loop(0, NFR * NRG, time_frg, 0)

        def out_row(r, carry2):
            rg16 = (r // LANES) * LANES
            rmod = jnp.full((LANES,), r % LANES, jnp.int32)
            for f in range(NFR):
                tv = tau_s[pl.ds(f * NINST + rg16, LANES)]
                tt = tv[rmod]
                for cg in range(LST // LANES):
                    c0 = f * LST + cg * LANES
                    z = buf[r, pl.ds(c0, LANES)]
                    ti = tau_i[pl.ds(c0, LANES)]
                    buf[r, pl.ds(c0, LANES)] = _relu(z - ti) * _relu(z - tt)
            return carry2

        lax.fori_loop(0, NINST, out_row, 0)

    pltpu.async_copy(hbm_at(0), buf0, sin0)

    def pair(j, carry):
        ci0 = 2 * j
        ci1 = ci0 + 1

        @pl.when(j > 0)
        def _():
            pltpu.make_async_copy(buf1, out_at(ci1 - 2), sout1).wait()

        pltpu.async_copy(hbm_at(ci1), buf1, sin1)
        pltpu.make_async_copy(hbm_at(ci0), buf0, sin0).wait()
        compute(buf0)
        pltpu.async_copy(buf0, out_at(ci0), sout0)

        pltpu.make_async_copy(hbm_at(ci1), buf1, sin1).wait()
        compute(buf1)
        pltpu.async_copy(buf1, out_at(ci1), sout1)

        @pl.when(ci0 + 2 < NCHUNK)
        def _():
            pltpu.make_async_copy(buf0, out_at(ci0), sout0).wait()
            pltpu.async_copy(hbm_at(ci0 + 2), buf0, sin0)

        return carry

    lax.fori_loop(0, NCHUNK // 2, pair, 0)
    pltpu.make_async_copy(buf0, out_at(NCHUNK - 2), sout0).wait()
    pltpu.make_async_copy(buf1, out_at(NCHUNK - 1), sout1).wait()


@jax.jit
def kernel(midis_out):
    mesh = plsc.VectorSubcoreMesh(core_axis_name="c", subcore_axis_name="s")
    fn = pl.kernel(
        _sc_body,
        out_type=jax.ShapeDtypeStruct((BATCH, NINST, TIME), jnp.float32),
        mesh=mesh,
        compiler_params=pltpu.CompilerParams(needs_layout_passes=False),
        scratch_types=[
            pltpu.VMEM((NINST, W), jnp.float32),
            pltpu.VMEM((NINST, W), jnp.float32),
            pltpu.VMEM((W,), jnp.float32),
            pltpu.VMEM((NFR * NINST,), jnp.float32),
            pltpu.SemaphoreType.DMA,
            pltpu.SemaphoreType.DMA,
            pltpu.SemaphoreType.DMA,
            pltpu.SemaphoreType.DMA,
        ],
    )
    return fn(midis_out)
